# baseline (speedup 1.0000x reference)
import jax
import jax.numpy as jnp
from jax import lax
from jax.experimental import pallas as pl
from jax.experimental.pallas import tpu as pltpu
from jax.experimental.pallas import tpu_sc as plsc

N = 10000
E = 160000
D = 128
NC, NS, L = 2, 16, 16
NW = NC * NS
NPAD = 10240
EB = 128
BLK_PER_TILE = 40
NBLKP = NW * BLK_PER_TILE
EPAD = NBLKP * EB
ROWS_PER_TILE = NPAD // NW
SELF_CB = 80
DEG_SL = NPAD // NS

_mesh = plsc.VectorSubcoreMesh(core_axis_name="c", subcore_axis_name="s")

NCT = (N + 127) // 128
AWQ = NCT * N
_BRC = 2000


NH = N // 2
_BRC = 40


def _combine_body(w_ref, alo_ref, ahi_ref, aw_ref):
    wv = w_ref[...]
    m = jnp.max(wv)
    e = jnp.exp(wv - m)
    wn = e / jnp.sum(e)
    w0 = wn[0, 0]
    w1 = wn[0, 1]
    for chi in range(NCT):
        w = min(128, N - chi * 128)
        sl = slice(chi * 128, chi * 128 + w)
        vlo = w0 * alo_ref[0, :, sl] + w1 * alo_ref[1, :, sl]
        vhi = w0 * ahi_ref[0, :, sl] + w1 * ahi_ref[1, :, sl]
        blo = lax.bitcast_convert_type(
            vlo.astype(jnp.bfloat16), jnp.uint16).astype(jnp.uint32)
        bhi = lax.bitcast_convert_type(
            vhi.astype(jnp.bfloat16), jnp.uint16).astype(jnp.uint32)
        aw_ref[chi, :, 0:w] = lax.bitcast_convert_type(
            blo | (bhi << 16), jnp.int32)


def _combine(wpad128, a_stack):
    return pl.pallas_call(
        _combine_body,
        grid=(NH // _BRC,),
        in_specs=[pl.BlockSpec((1, 128), lambda i: (0, 0)),
                  pl.BlockSpec((2, _BRC, N), lambda i: (0, i, 0)),
                  pl.BlockSpec((2, _BRC, N),
                               lambda i: (0, i + NH // _BRC, 0))],
        out_specs=pl.BlockSpec((NCT, _BRC, 128), lambda i: (0, i, 0)),
        out_shape=jax.ShapeDtypeStruct((NCT, NH, 128), jnp.int32),
    )(wpad128, a_stack, a_stack)


def _edge_weights_body(aw, rows2, cols2,
                       ew2, deg_h,
                       rbig, cbig, fb, sb_, gg, ewf, zb, dacc, semg0, semg1):
    cid = lax.axis_index("c")
    sid = lax.axis_index("s")
    wid = sid * NC + cid
    tb = wid * BLK_PER_TILE

    pltpu.sync_copy(rows2.at[pl.ds(tb, BLK_PER_TILE)], rbig)
    pltpu.sync_copy(cols2.at[pl.ds(tb, BLK_PER_TILE)], cbig)

    def zloop(j, _):
        zb[pl.ds(j * L, L)] = jnp.zeros((L,), jnp.float32)
        return 0
    lax.fori_loop(0, DEG_SL // L, zloop, 0)
    pltpu.sync_copy(zb, dacc.at[pl.ds(sid * DEG_SL, DEG_SL)])
    plsc.subcore_barrier()

    sems = (semg0, semg1)

    def fidx(b, p):
        def fx(j, _):
            sl = pl.ds(j * L, L)
            rv = rbig[b, sl]
            cv = cbig[b, sl]
            chi = lax.shift_right_logical(cv, 7)
            clo = jnp.bitwise_and(cv, 127)
            one = jnp.int32(1)
            zero = jnp.int32(0)
            losel = rv < NH
            u = jnp.where(losel, rv, rv - NH)
            fb[p, sl] = (chi * NH + u) * 128 + clo
            sb_[p, sl] = jnp.where(losel, one, zero)
            return 0
        lax.fori_loop(0, EB // L, fx, 0)

    def decode(p):
        def dx(j, _):
            sl = pl.ds(j * L, L)
            g = gg[p, sl]
            b = jnp.where(sb_[p, sl] == 1,
                          jnp.bitwise_and(g, 65535),
                          jnp.bitwise_and(lax.shift_right_logical(g, 16),
                                          65535))
            e = jnp.bitwise_and(lax.shift_right_logical(b, 7), 255)
            mant = jnp.bitwise_and(b, 127)
            ef = e.astype(jnp.float32)
            mf = mant.astype(jnp.float32)
            ewf[p, sl] = (jnp.exp(0.6931471805599453 * (ef - 127.0))
                          * (1.0 + mf * 0.0078125))
            return 0
        lax.fori_loop(0, EB // L, dx, 0)

    for p in range(2):
        fidx(p, p)
        pltpu.async_copy(aw.at[fb.at[p]], gg.at[p], sems[p])

    def blk(g, _):
        for p in range(2):
            b = g * 2 + p
            pltpu.make_async_copy(aw.at[fb.at[p]], gg.at[p], sems[p]).wait()
            decode(p)
            pltpu.sync_copy(ewf.at[p], ew2.at[tb + b])
            pltpu.sync_copy(ewf.at[p], dacc.at[cbig.at[b]], add=True)

            @pl.when(b + 2 < BLK_PER_TILE)
            def _():
                fidx(b + 2, p)
                pltpu.async_copy(aw.at[fb.at[p]], gg.at[p], sems[p])
        return 0
    lax.fori_loop(0, BLK_PER_TILE // 2, blk, 0)

    plsc.subcore_barrier()
    pltpu.sync_copy(dacc.at[pl.ds(sid * DEG_SL, DEG_SL)],
                    deg_h.at[pl.ds(cid * NPAD + sid * DEG_SL, DEG_SL)])


_edge_weights = pl.kernel(
    _edge_weights_body,
    out_type=(jax.ShapeDtypeStruct((NBLKP, EB), jnp.float32),
              jax.ShapeDtypeStruct((NC * NPAD,), jnp.float32)),
    mesh=_mesh,
    scratch_types=[
        pltpu.VMEM((BLK_PER_TILE, EB), jnp.int32),
        pltpu.VMEM((BLK_PER_TILE, EB), jnp.int32),
        pltpu.VMEM((2, EB), jnp.int32),
        pltpu.VMEM((2, EB), jnp.int32),
        pltpu.VMEM((2, EB), jnp.int32),
        pltpu.VMEM((2, EB), jnp.float32),
        pltpu.VMEM((DEG_SL,), jnp.float32),
        pltpu.VMEM_SHARED((NPAD,), jnp.float32),
        pltpu.SemaphoreType.DMA,
        pltpu.SemaphoreType.DMA,
    ],
)


def _matmul_body(x_ref, w_ref, h_ref):
    h_ref[...] = lax.dot_general(
        x_ref[...], w_ref[...],
        dimension_numbers=(((1,), (1,)), ((), ())),
        preferred_element_type=jnp.float32,
        precision=lax.Precision.HIGHEST)


_MB = 512


def _matmul(xpad, w):
    return pl.pallas_call(
        _matmul_body,
        grid=(NPAD // _MB,),
        in_specs=[pl.BlockSpec((_MB, D), lambda i: (i, 0)),
                  pl.BlockSpec((D, D), lambda i: (0, 0))],
        out_specs=pl.BlockSpec((_MB, D), lambda i: (i, 0)),
        out_shape=jax.ShapeDtypeStruct((NPAD, D), jnp.float32),
    )(xpad, w)


def _dis_body(degp_ref, dis_ref):
    deg = degp_ref[0] + degp_ref[1] + 1.0
    pos = deg > 0
    safe = jnp.where(pos, deg, 1.0)
    dis_ref[...] = jnp.where(pos, lax.rsqrt(safe), 0.0)


def _dis(degp):
    return pl.pallas_call(
        _dis_body,
        out_shape=jax.ShapeDtypeStruct((NPAD // D, D), jnp.float32),
    )(degp)


def _scatter_body(rows2, cols2, ew2, dis_h, h_h,
                  acc_h,
                  rbig, cbig, ewbig, drb, dcb, nrm, sdis, hb, idb, accsh,
                  semh, semr, semc):
    cid = lax.axis_index("c")
    sid = lax.axis_index("s")
    wid = sid * NC + cid
    tb = wid * BLK_PER_TILE

    pltpu.sync_copy(rows2.at[pl.ds(tb, BLK_PER_TILE)], rbig)
    pltpu.sync_copy(cols2.at[pl.ds(tb, BLK_PER_TILE)], cbig)
    pltpu.sync_copy(ew2.at[pl.ds(tb, BLK_PER_TILE)], ewbig)

    def z(i, _):
        r = i // (D // L)
        mm = i % (D // L)
        hb[0, r, pl.ds(mm * L, L)] = jnp.zeros((L,), jnp.float32)
        return 0
    lax.fori_loop(0, EB * (D // L), z, 0)

    def zc(k, _):
        pltpu.sync_copy(hb.at[0], accsh.at[pl.ds(sid * DEG_SL + k * EB, EB)])
        return 0
    lax.fori_loop(0, DEG_SL // EB, zc, 0)
    plsc.subcore_barrier()

    def selfc(k, _):
        sb = wid * ROWS_PER_TILE + k * SELF_CB
        pltpu.sync_copy(h_h.at[pl.ds(sb, SELF_CB)], hb.at[0, pl.ds(0, SELF_CB)])
        pltpu.sync_copy(dis_h.at[pl.ds(sb, SELF_CB)], sdis)

        def mkid(j, _):
            idb[pl.ds(j * L, L)] = sb + j * L + lax.iota(jnp.int32, L)
            return 0
        lax.fori_loop(0, SELF_CB // L, mkid, 0)

        def scale(g, _):
            dv = sdis[pl.ds(g * L, L)]
            qv = dv * dv
            for r in range(L):
                q = qv[r]
                row = g * L + r
                for mm in range(D // L):
                    sl = pl.ds(mm * L, L)
                    hb[0, row, sl] = hb[0, row, sl] * q
            return 0
        lax.fori_loop(0, SELF_CB // L, scale, 0)

        pltpu.sync_copy(hb.at[0, pl.ds(0, SELF_CB)], accsh.at[idb], add=True)
        return 0
    lax.fori_loop(0, ROWS_PER_TILE // SELF_CB, selfc, 0)

    def issue(b, s):
        pltpu.async_copy(h_h.at[rbig.at[b]], hb.at[s], semh.at[s])
        pltpu.async_copy(dis_h.at[rbig.at[b]], drb.at[s], semr.at[s])
        pltpu.async_copy(dis_h.at[cbig.at[b]], dcb.at[s], semc.at[s])

    issue(0, 0)
    issue(1, 1)

    def blk(g, _):
        for s in range(2):
            b = g * 2 + s
            pltpu.make_async_copy(
                h_h.at[rbig.at[b]], hb.at[s], semh.at[s]).wait()
            pltpu.make_async_copy(
                dis_h.at[rbig.at[b]], drb.at[s], semr.at[s]).wait()
            pltpu.make_async_copy(
                dis_h.at[cbig.at[b]], dcb.at[s], semc.at[s]).wait()

            def nx(j, _):
                sl = pl.ds(j * L, L)
                nrm[sl] = drb[s, sl] * ewbig[b, sl] * dcb[s, sl]
                return 0
            lax.fori_loop(0, EB // L, nx, 0)

            def scale(g2, _):
                nv = nrm[pl.ds(g2 * L, L)]
                for r in range(L):
                    q = nv[r]
                    row = g2 * L + r
                    for mm in range(D // L):
                        sl = pl.ds(mm * L, L)
                        hb[s, row, sl] = hb[s, row, sl] * q
                return 0
            lax.fori_loop(0, EB // L, scale, 0)

            pltpu.sync_copy(hb.at[s], accsh.at[cbig.at[b]], add=True)

            @pl.when(b + 2 < BLK_PER_TILE)
            def _():
                issue(b + 2, s)
        return 0
    lax.fori_loop(0, BLK_PER_TILE // 2, blk, 0)

    plsc.subcore_barrier()
    pltpu.sync_copy(accsh.at[pl.ds(sid * DEG_SL, DEG_SL)],
                    acc_h.at[cid, pl.ds(sid * DEG_SL, DEG_SL)])


_scatter = pl.kernel(
    _scatter_body,
    out_type=jax.ShapeDtypeStruct((NC, NPAD, D), jnp.float32),
    mesh=_mesh,
    scratch_types=[
        pltpu.VMEM((BLK_PER_TILE, EB), jnp.int32),
        pltpu.VMEM((BLK_PER_TILE, EB), jnp.int32),
        pltpu.VMEM((BLK_PER_TILE, EB), jnp.float32),
        pltpu.VMEM((2, EB), jnp.float32),
        pltpu.VMEM((2, EB), jnp.float32),
        pltpu.VMEM((EB,), jnp.float32),
        pltpu.VMEM((SELF_CB,), jnp.float32),
        pltpu.VMEM((2, EB, D), jnp.float32),
        pltpu.VMEM((SELF_CB,), jnp.int32),
        pltpu.VMEM_SHARED((NPAD, D), jnp.float32),
        pltpu.SemaphoreType.DMA((2,)),
        pltpu.SemaphoreType.DMA((2,)),
        pltpu.SemaphoreType.DMA((2,)),
    ],
)


def _final_body(acc_ref, bias_ref, out_ref):
    out_ref[...] = acc_ref[0] + acc_ref[1] + bias_ref[...]


_FB = 400


def _final(acc, bias2d):
    return pl.pallas_call(
        _final_body,
        grid=(N // _FB,),
        in_specs=[pl.BlockSpec((NC, _FB, D), lambda i: (0, i, 0)),
                  pl.BlockSpec((1, D), lambda i: (0, 0))],
        out_specs=pl.BlockSpec((_FB, D), lambda i: (i, 0)),
        out_shape=jax.ShapeDtypeStruct((N, D), jnp.float32),
    )(acc, bias2d)


def kernel(x, edge_index, A_stack, weights, W, bias):
    npd = EPAD - E
    it = lax.iota(jnp.int32, npd)
    rows_p = jnp.concatenate([edge_index[0], it % N])
    cols_p = jnp.concatenate([edge_index[1], N + (it % 112)])
    rows2 = rows_p.reshape(NBLKP, EB)
    cols2 = cols_p.reshape(NBLKP, EB)
    wpad128 = jnp.concatenate(
        [weights.astype(jnp.float32),
         jnp.full((126,), -jnp.inf, jnp.float32)]).reshape(1, 128)
    xpad = jnp.concatenate(
        [x, jnp.zeros((NPAD - N, D), jnp.float32)], axis=0)

    aw = _combine(wpad128, A_stack).reshape(NCT * NH * 128)
    ew2, degp = _edge_weights(aw, rows2, cols2)
    h = _matmul(xpad, W)
    dis = _dis(degp.reshape(NC, NPAD // D, D)).reshape(NPAD)
    acc = _scatter(rows2, cols2, ew2, dis, h)
    return _final(acc, bias.reshape(1, D))


# speedup vs baseline: 1.0091x; 1.0091x over previous
import jax
import jax.numpy as jnp
from jax import lax
from jax.experimental import pallas as pl
from jax.experimental.pallas import tpu as pltpu
from jax.experimental.pallas import tpu_sc as plsc

N = 10000
E = 160000
D = 128
NC, NS, L = 2, 16, 16
NW = NC * NS
NPAD = 10240
EB = 128
BLK_PER_TILE = 40
NBLKP = NW * BLK_PER_TILE
EPAD = NBLKP * EB
ROWS_PER_TILE = NPAD // NW
SELF_CB = 80
DEG_SL = NPAD // NS

_mesh = plsc.VectorSubcoreMesh(core_axis_name="c", subcore_axis_name="s")

NCT = (N + 127) // 128
AWQ = NCT * N
_BRC = 2000


NH = N // 2
_BRC = 40


def _combine_body(w_ref, alo_ref, ahi_ref, aw_ref):
    wv = w_ref[...]
    m = jnp.max(wv)
    e = jnp.exp(wv - m)
    wn = e / jnp.sum(e)
    w0 = wn[0, 0]
    w1 = wn[0, 1]
    for chi in range(NCT):
        w = min(128, N - chi * 128)
        sl = slice(chi * 128, chi * 128 + w)
        vlo = w0 * alo_ref[0, :, sl] + w1 * alo_ref[1, :, sl]
        vhi = w0 * ahi_ref[0, :, sl] + w1 * ahi_ref[1, :, sl]
        blo = lax.bitcast_convert_type(
            vlo.astype(jnp.bfloat16), jnp.uint16).astype(jnp.uint32)
        bhi = lax.bitcast_convert_type(
            vhi.astype(jnp.bfloat16), jnp.uint16).astype(jnp.uint32)
        aw_ref[chi, :, 0:w] = lax.bitcast_convert_type(
            blo | (bhi << 16), jnp.int32)


def _combine(wpad128, a_stack):
    return pl.pallas_call(
        _combine_body,
        grid=(NH // _BRC,),
        in_specs=[pl.BlockSpec((1, 128), lambda i: (0, 0)),
                  pl.BlockSpec((2, _BRC, N), lambda i: (0, i, 0)),
                  pl.BlockSpec((2, _BRC, N),
                               lambda i: (0, i + NH // _BRC, 0))],
        out_specs=pl.BlockSpec((NCT, _BRC, 128), lambda i: (0, i, 0)),
        out_shape=jax.ShapeDtypeStruct((NCT, NH, 128), jnp.int32),
    )(wpad128, a_stack, a_stack)


def _edge_weights_body(aw, rows2, cols2,
                       ew2, deg_h,
                       rbig, cbig, fb, sb_, gg, ewf, zb, dacc, semg0, semg1):
    cid = lax.axis_index("c")
    sid = lax.axis_index("s")
    wid = sid * NC + cid
    tb = wid * BLK_PER_TILE

    pltpu.sync_copy(rows2.at[pl.ds(tb, BLK_PER_TILE)], rbig)
    pltpu.sync_copy(cols2.at[pl.ds(tb, BLK_PER_TILE)], cbig)

    def zloop(j, _):
        zb[pl.ds(j * L, L)] = jnp.zeros((L,), jnp.float32)
        return 0
    lax.fori_loop(0, DEG_SL // L, zloop, 0)
    pltpu.sync_copy(zb, dacc.at[pl.ds(sid * DEG_SL, DEG_SL)])
    plsc.subcore_barrier()

    sems = (semg0, semg1)

    def fidx(b, p):
        def fx(j, _):
            sl = pl.ds(j * L, L)
            rv = rbig[b, sl]
            cv = cbig[b, sl]
            chi = lax.shift_right_logical(cv, 7)
            clo = jnp.bitwise_and(cv, 127)
            one = jnp.int32(1)
            zero = jnp.int32(0)
            losel = rv < NH
            u = jnp.where(losel, rv, rv - NH)
            fb[p, sl] = (chi * NH + u) * 128 + clo
            sb_[p, sl] = jnp.where(losel, one, zero)
            return 0
        lax.fori_loop(0, EB // L, fx, 0)

    def decode(p):
        def dx(j, _):
            sl = pl.ds(j * L, L)
            g = gg[p, sl]
            b = jnp.where(sb_[p, sl] == 1,
                          jnp.bitwise_and(g, 65535),
                          jnp.bitwise_and(lax.shift_right_logical(g, 16),
                                          65535))
            e = jnp.bitwise_and(lax.shift_right_logical(b, 7), 255)
            mant = jnp.bitwise_and(b, 127)
            ef = e.astype(jnp.float32)
            mf = mant.astype(jnp.float32)
            ewf[p, sl] = (jnp.exp(0.6931471805599453 * (ef - 127.0))
                          * (1.0 + mf * 0.0078125))
            return 0
        lax.fori_loop(0, EB // L, dx, 0)

    for p in range(2):
        fidx(p, p)
        pltpu.async_copy(aw.at[fb.at[p]], gg.at[p], sems[p])

    def blk(g, _):
        for p in range(2):
            b = g * 2 + p
            pltpu.make_async_copy(aw.at[fb.at[p]], gg.at[p], sems[p]).wait()
            decode(p)
            pltpu.sync_copy(ewf.at[p], ew2.at[tb + b])
            pltpu.sync_copy(ewf.at[p], dacc.at[cbig.at[b]], add=True)

            @pl.when(b + 2 < BLK_PER_TILE)
            def _():
                fidx(b + 2, p)
                pltpu.async_copy(aw.at[fb.at[p]], gg.at[p], sems[p])
        return 0
    lax.fori_loop(0, BLK_PER_TILE // 2, blk, 0)

    plsc.subcore_barrier()
    pltpu.sync_copy(dacc.at[pl.ds(sid * DEG_SL, DEG_SL)],
                    deg_h.at[cid, pl.ds(sid * DEG_SL, DEG_SL)])


_edge_weights = pl.kernel(
    _edge_weights_body,
    out_type=(jax.ShapeDtypeStruct((NBLKP, EB), jnp.float32),
              jax.ShapeDtypeStruct((NC, NPAD), jnp.float32)),
    mesh=_mesh,
    scratch_types=[
        pltpu.VMEM((BLK_PER_TILE, EB), jnp.int32),
        pltpu.VMEM((BLK_PER_TILE, EB), jnp.int32),
        pltpu.VMEM((2, EB), jnp.int32),
        pltpu.VMEM((2, EB), jnp.int32),
        pltpu.VMEM((2, EB), jnp.int32),
        pltpu.VMEM((2, EB), jnp.float32),
        pltpu.VMEM((DEG_SL,), jnp.float32),
        pltpu.VMEM_SHARED((NPAD,), jnp.float32),
        pltpu.SemaphoreType.DMA,
        pltpu.SemaphoreType.DMA,
    ],
)


def _matmul_body(x_ref, w_ref, h_ref):
    h_ref[...] = lax.dot_general(
        x_ref[...], w_ref[...],
        dimension_numbers=(((1,), (1,)), ((), ())),
        preferred_element_type=jnp.float32,
        precision=lax.Precision.HIGHEST)


_MB = 512


def _matmul(xpad, w):
    return pl.pallas_call(
        _matmul_body,
        grid=(NPAD // _MB,),
        in_specs=[pl.BlockSpec((_MB, D), lambda i: (i, 0)),
                  pl.BlockSpec((D, D), lambda i: (0, 0))],
        out_specs=pl.BlockSpec((_MB, D), lambda i: (i, 0)),
        out_shape=jax.ShapeDtypeStruct((NPAD, D), jnp.float32),
    )(xpad, w)


def _dis_body(degp_ref, dis_ref):
    deg = degp_ref[0] + degp_ref[1] + 1.0
    pos = deg > 0
    safe = jnp.where(pos, deg, 1.0)
    dis_ref[...] = jnp.where(pos, lax.rsqrt(safe), 0.0)


def _dis(degp):
    return pl.pallas_call(
        _dis_body,
        out_shape=jax.ShapeDtypeStruct((NPAD // D, D), jnp.float32),
    )(degp)


def _scatter_body(rows2, cols2, ew2, dis_h, h_h,
                  acc_h,
                  rbig, cbig, ewbig, drb, dcb, nrm, sdis, hb, idb, accsh,
                  semh0, semh1, semr0, semr1, semc0, semc1):
    cid = lax.axis_index("c")
    sid = lax.axis_index("s")
    wid = sid * NC + cid
    tb = wid * BLK_PER_TILE

    pltpu.sync_copy(rows2.at[pl.ds(tb, BLK_PER_TILE)], rbig)
    pltpu.sync_copy(cols2.at[pl.ds(tb, BLK_PER_TILE)], cbig)
    pltpu.sync_copy(ew2.at[pl.ds(tb, BLK_PER_TILE)], ewbig)

    def z(i, _):
        r = i // (D // L)
        mm = i % (D // L)
        hb[0, r, pl.ds(mm * L, L)] = jnp.zeros((L,), jnp.float32)
        return 0
    lax.fori_loop(0, EB * (D // L), z, 0)

    def zc(k, _):
        pltpu.sync_copy(hb.at[0], accsh.at[pl.ds(sid * DEG_SL + k * EB, EB)])
        return 0
    lax.fori_loop(0, DEG_SL // EB, zc, 0)
    plsc.subcore_barrier()

    def selfc(k, _):
        sb = wid * ROWS_PER_TILE + k * SELF_CB
        pltpu.sync_copy(h_h.at[pl.ds(sb, SELF_CB)], hb.at[0, pl.ds(0, SELF_CB)])
        pltpu.sync_copy(dis_h.at[pl.ds(sb, SELF_CB)], sdis)

        def mkid(j, _):
            idb[pl.ds(j * L, L)] = sb + j * L + lax.iota(jnp.int32, L)
            return 0
        lax.fori_loop(0, SELF_CB // L, mkid, 0)

        def scale(g, _):
            dv = sdis[pl.ds(g * L, L)]
            qv = dv * dv
            for r in range(L):
                q = qv[r]
                row = g * L + r
                for mm in range(D // L):
                    sl = pl.ds(mm * L, L)
                    hb[0, row, sl] = hb[0, row, sl] * q
            return 0
        lax.fori_loop(0, SELF_CB // L, scale, 0)

        pltpu.sync_copy(hb.at[0, pl.ds(0, SELF_CB)], accsh.at[idb], add=True)
        return 0
    lax.fori_loop(0, ROWS_PER_TILE // SELF_CB, selfc, 0)

    semh = (semh0, semh1)
    semr = (semr0, semr1)
    semc = (semc0, semc1)

    def issue(b, s):
        pltpu.async_copy(h_h.at[rbig.at[b]], hb.at[s], semh[s])
        pltpu.async_copy(dis_h.at[rbig.at[b]], drb.at[s], semr[s])
        pltpu.async_copy(dis_h.at[cbig.at[b]], dcb.at[s], semc[s])

    issue(0, 0)
    issue(1, 1)

    def blk(g, _):
        for s in range(2):
            b = g * 2 + s
            pltpu.make_async_copy(
                h_h.at[rbig.at[b]], hb.at[s], semh[s]).wait()
            pltpu.make_async_copy(
                dis_h.at[rbig.at[b]], drb.at[s], semr[s]).wait()
            pltpu.make_async_copy(
                dis_h.at[cbig.at[b]], dcb.at[s], semc[s]).wait()

            def nx(j, _):
                sl = pl.ds(j * L, L)
                nrm[sl] = drb[s, sl] * ewbig[b, sl] * dcb[s, sl]
                return 0
            lax.fori_loop(0, EB // L, nx, 0)

            def scale(g2, _):
                nv = nrm[pl.ds(g2 * L, L)]
                for r in range(L):
                    q = nv[r]
                    row = g2 * L + r
                    for mm in range(D // L):
                        sl = pl.ds(mm * L, L)
                        hb[s, row, sl] = hb[s, row, sl] * q
                return 0
            lax.fori_loop(0, EB // L, scale, 0)

            pltpu.sync_copy(hb.at[s], accsh.at[cbig.at[b]], add=True)

            @pl.when(b + 2 < BLK_PER_TILE)
            def _():
                issue(b + 2, s)
        return 0
    lax.fori_loop(0, BLK_PER_TILE // 2, blk, 0)

    plsc.subcore_barrier()
    pltpu.sync_copy(accsh.at[pl.ds(sid * DEG_SL, DEG_SL)],
                    acc_h.at[cid, pl.ds(sid * DEG_SL, DEG_SL)])


_scatter = pl.kernel(
    _scatter_body,
    out_type=jax.ShapeDtypeStruct((NC, NPAD, D), jnp.float32),
    mesh=_mesh,
    scratch_types=[
        pltpu.VMEM((BLK_PER_TILE, EB), jnp.int32),
        pltpu.VMEM((BLK_PER_TILE, EB), jnp.int32),
        pltpu.VMEM((BLK_PER_TILE, EB), jnp.float32),
        pltpu.VMEM((2, EB), jnp.float32),
        pltpu.VMEM((2, EB), jnp.float32),
        pltpu.VMEM((EB,), jnp.float32),
        pltpu.VMEM((SELF_CB,), jnp.float32),
        pltpu.VMEM((2, EB, D), jnp.float32),
        pltpu.VMEM((SELF_CB,), jnp.int32),
        pltpu.VMEM_SHARED((NPAD, D), jnp.float32),
        pltpu.SemaphoreType.DMA,
        pltpu.SemaphoreType.DMA,
        pltpu.SemaphoreType.DMA,
        pltpu.SemaphoreType.DMA,
        pltpu.SemaphoreType.DMA,
        pltpu.SemaphoreType.DMA,
    ],
)


def _final_body(acc_ref, bias_ref, out_ref):
    out_ref[...] = acc_ref[0] + acc_ref[1] + bias_ref[...]


_FB = 400


def _final(acc, bias2d):
    return pl.pallas_call(
        _final_body,
        grid=(N // _FB,),
        in_specs=[pl.BlockSpec((NC, _FB, D), lambda i: (0, i, 0)),
                  pl.BlockSpec((1, D), lambda i: (0, 0))],
        out_specs=pl.BlockSpec((_FB, D), lambda i: (i, 0)),
        out_shape=jax.ShapeDtypeStruct((N, D), jnp.float32),
    )(acc, bias2d)


def kernel(x, edge_index, A_stack, weights, W, bias):
    npd = EPAD - E
    it = lax.iota(jnp.int32, npd)
    rows_p = jnp.concatenate([edge_index[0], it % N])
    cols_p = jnp.concatenate([edge_index[1], N + (it % 112)])
    rows2 = rows_p.reshape(NBLKP, EB)
    cols2 = cols_p.reshape(NBLKP, EB)
    wpad128 = jnp.concatenate(
        [weights.astype(jnp.float32),
         jnp.full((126,), -jnp.inf, jnp.float32)]).reshape(1, 128)
    xpad = jnp.concatenate(
        [x, jnp.zeros((NPAD - N, D), jnp.float32)], axis=0)

    aw = _combine(wpad128, A_stack).reshape(NCT * NH * 128)
    ew2, degp = _edge_weights(aw, rows2, cols2)
    h = _matmul(xpad, W)
    dis = _dis(degp.reshape(NC, NPAD // D, D)).reshape(NPAD)
    acc = _scatter(rows2, cols2, ew2, dis, h)
    return _final(acc, bias.reshape(1, D))
